# baseline (speedup 1.0000x reference)
import jax
import jax.numpy as jnp
from jax.experimental import pallas as pl
from jax.experimental.pallas import tpu as pltpu


def kernel(input, table):
    B, S, D = input.shape
    BB = 64
    NDMA = B // BB

    def body(table_ref, out_hbm, buf, fill_sem, out_sem):
        buf[0] = table_ref[:S, :]
        k = 1
        while k < BB:
            cp = pltpu.make_async_copy(
                buf.at[pl.ds(0, k)], buf.at[pl.ds(k, k)], fill_sem)
            cp.start()
            cp.wait()
            k *= 2
        for i in range(NDMA):
            pltpu.make_async_copy(
                buf, out_hbm.at[pl.ds(i * BB, BB)], out_sem).start()
        for i in range(NDMA):
            pltpu.make_async_copy(
                buf, out_hbm.at[pl.ds(i * BB, BB)], out_sem).wait()

    out = pl.pallas_call(
        body,
        in_specs=[pl.BlockSpec(memory_space=pltpu.VMEM)],
        out_specs=pl.BlockSpec(memory_space=pl.ANY),
        out_shape=jax.ShapeDtypeStruct((B, S, D), jnp.float32),
        scratch_shapes=[
            pltpu.VMEM((BB, S, D), jnp.float32),
            pltpu.SemaphoreType.DMA,
            pltpu.SemaphoreType.DMA,
        ],
    )(table)
    return out


# speedup vs baseline: 1.6491x; 1.6491x over previous
import jax
import jax.numpy as jnp
from jax.experimental import pallas as pl


def kernel(input, table):
    B, S, D = input.shape
    V = table.shape[0]
    F = S * D
    BB = 64

    tbl2 = jnp.reshape(table, (1, V * D))

    def body(t_ref, out_ref):
        emb = t_ref[:, :F]
        out_ref[...] = jnp.broadcast_to(emb, (BB, F))

    out2 = pl.pallas_call(
        body,
        grid=(B // BB,),
        in_specs=[pl.BlockSpec((1, V * D), lambda i: (0, 0))],
        out_specs=pl.BlockSpec((BB, F), lambda i: (i, 0)),
        out_shape=jax.ShapeDtypeStruct((B, F), jnp.float32),
    )(tbl2)
    return jnp.reshape(out2, (B, S, D))
